# baseline (speedup 1.0000x reference)
import functools

import jax
import jax.numpy as jnp
from jax import lax
from jax.experimental import pallas as pl
from jax.experimental.pallas import tpu as pltpu
from jax.experimental.pallas import tpu_sc as plsc

B = 16384
V = 100000
D1 = 64
D2 = 128
NC = 2
NS = 16
NW = NC * NS
B_PER_W = B // NW
CHUNK2 = 64
NCHUNK2 = B_PER_W // CHUNK2
R_PER_W = D1 // NW
IDX_CHUNK = 2048
NIDX = B // IDX_CHUNK
L = 16

_mesh = plsc.VectorSubcoreMesh(core_axis_name="c", subcore_axis_name="s")


@functools.partial(
    pl.kernel,
    mesh=_mesh,
    compiler_params=pltpu.CompilerParams(
        use_tc_tiling_on_sc=True, needs_layout_passes=False,
        internal_scratch_in_bytes=65536),
    out_type=(
        jax.ShapeDtypeStruct((D1, B), jnp.float32),
        jax.ShapeDtypeStruct((B, D2), jnp.float32),
    ),
    scratch_types=[
        pltpu.VMEM((B_PER_W,), jnp.int32),
        pltpu.VMEM((CHUNK2, D2), jnp.float32),
        pltpu.VMEM((CHUNK2, D2), jnp.float32),
        pltpu.VMEM((1, V), jnp.float32),
        pltpu.VMEM((IDX_CHUNK,), jnp.int32),
        pltpu.VMEM((IDX_CHUNK,), jnp.int32),
        pltpu.VMEM((1, IDX_CHUNK), jnp.float32),
        pltpu.VMEM((1, IDX_CHUNK), jnp.float32),
        pltpu.SemaphoreType.DMA,
        pltpu.SemaphoreType.DMA,
        pltpu.SemaphoreType.DMA,
        pltpu.SemaphoreType.DMA,
        pltpu.SemaphoreType.DMA,
        pltpu.SemaphoreType.DMA,
    ],
    name="sc_dual_gather",
)
def _dual_gather(ids_hbm, w1t_hbm, w2_hbm, out1t_hbm, out2_hbm,
                 idx_v, rows2a_v, rows2b_v, wrow_v,
                 idxga_v, idxgb_v, oga_v, ogb_v,
                 sem2a, sem2b, semr, semi, semo, semw):
    wid = lax.axis_index("s") * NC + lax.axis_index("c")
    base = wid * B_PER_W
    j0 = wid * R_PER_W

    hrow = pltpu.async_copy(w1t_hbm.at[pl.ds(j0, 1)], wrow_v, semr)
    pltpu.sync_copy(ids_hbm.at[pl.ds(base, B_PER_W)], idx_v)

    bufs = (rows2a_v, rows2b_v)
    sems = (sem2a, sem2b)

    def _gather2(j):
        return pltpu.async_copy(
            w2_hbm.at[idx_v.at[pl.ds(j * CHUNK2, CHUNK2)]],
            bufs[j % 2], sems[j % 2])

    inflight = [_gather2(0), _gather2(1)]

    idxg = (idxga_v, idxgb_v)
    og = (oga_v, ogb_v)

    def _fetch_idx(c, p):
        return pltpu.async_copy(
            ids_hbm.at[pl.ds(c * IDX_CHUNK, IDX_CHUNK)], idxg[p], semi)

    hidx = [_fetch_idx(0, 0), None]
    row_ref = wrow_v.at[0]
    og_w = [None, None]
    w2w = [None, None]

    hrow.wait()
    for r in range(R_PER_W):
        j = j0 + r
        if r > 0:
            hrow.wait()
        for c in range(NIDX):
            p = c % 2
            hidx[p].wait()
            if c + 1 < NIDX:
                hidx[(c + 1) % 2] = _fetch_idx(c + 1, (c + 1) % 2)
            elif r + 1 < R_PER_W:
                hidx[0] = _fetch_idx(0, 0)
            if og_w[p] is not None:
                og_w[p].wait()
            if r == 0:
                inflight[p].wait()
                w2w[p] = pltpu.async_copy(
                    bufs[p], out2_hbm.at[pl.ds(base + c * CHUNK2, CHUNK2)],
                    semw)
            ochunk = og[p].at[0]
            ichunk = idxg[p]

            @plsc.parallel_loop(0, IDX_CHUNK // L, unroll=16)
            def _grp(g):
                iv = ichunk[pl.ds(g * L, L)]
                ochunk[pl.ds(g * L, L)] = plsc.load_gather(row_ref, [iv])

            og_w[p] = pltpu.async_copy(
                og[p],
                out1t_hbm.at[pl.ds(j, 1), pl.ds(c * IDX_CHUNK, IDX_CHUNK)],
                semo)
            if r == 0 and c + 2 < NCHUNK2:
                w2w[p].wait()
                w2w[p] = None
                inflight[p] = _gather2(c + 2)
        if r + 1 < R_PER_W:
            hrow = pltpu.async_copy(
                w1t_hbm.at[pl.ds(j0 + r + 1, 1)], wrow_v, semr)
    for h in og_w + w2w:
        if h is not None:
            h.wait()


def kernel(instance_ids, W_instance, W_backgrounds):
    ids = jnp.squeeze(instance_ids).astype(jnp.int32)
    out1t, out2 = _dual_gather(ids, W_instance.T, W_backgrounds)
    return (out1t.T, out2)


# speedup vs baseline: 1.0020x; 1.0020x over previous
import functools

import jax
import jax.numpy as jnp
from jax import lax
from jax.experimental import pallas as pl
from jax.experimental.pallas import tpu as pltpu
from jax.experimental.pallas import tpu_sc as plsc

B = 16384
V = 100000
D1 = 64
D2 = 128
NC = 2
NS = 16
NW = NC * NS
B_PER_W = B // NW
CHUNK2 = 64
NCHUNK2 = B_PER_W // CHUNK2
R_PER_W = D1 // NW
IDX_CHUNK = 2048
NIDX = B // IDX_CHUNK
L = 16

_mesh = plsc.VectorSubcoreMesh(core_axis_name="c", subcore_axis_name="s")


@functools.partial(
    pl.kernel,
    mesh=_mesh,
    compiler_params=pltpu.CompilerParams(
        use_tc_tiling_on_sc=True, needs_layout_passes=False,
        internal_scratch_in_bytes=65536),
    out_type=(
        jax.ShapeDtypeStruct((D1, B), jnp.float32),
        jax.ShapeDtypeStruct((B, D2), jnp.float32),
    ),
    scratch_types=[
        pltpu.VMEM((B_PER_W,), jnp.int32),
        pltpu.VMEM((CHUNK2, D2), jnp.float32),
        pltpu.VMEM((CHUNK2, D2), jnp.float32),
        pltpu.VMEM((1, V), jnp.float32),
        pltpu.VMEM((IDX_CHUNK,), jnp.int32),
        pltpu.VMEM((IDX_CHUNK,), jnp.int32),
        pltpu.VMEM((1, IDX_CHUNK), jnp.float32),
        pltpu.VMEM((1, IDX_CHUNK), jnp.float32),
        pltpu.SemaphoreType.DMA,
        pltpu.SemaphoreType.DMA,
        pltpu.SemaphoreType.DMA,
        pltpu.SemaphoreType.DMA,
        pltpu.SemaphoreType.DMA,
        pltpu.SemaphoreType.DMA,
    ],
    name="sc_dual_gather",
)
def _dual_gather(ids_hbm, w1t_hbm, w2_hbm, out1t_hbm, out2_hbm,
                 idx_v, rows2a_v, rows2b_v, wrow_v,
                 idxga_v, idxgb_v, oga_v, ogb_v,
                 sem2a, sem2b, semr, semi, semo, semw):
    wid = lax.axis_index("s") * NC + lax.axis_index("c")
    base = wid * B_PER_W
    j0 = wid * R_PER_W
    VQ = 25088

    def _fetch_row(j):
        return [
            pltpu.async_copy(
                w1t_hbm.at[pl.ds(j, 1), pl.ds(q * VQ, min(VQ, V - q * VQ))],
                wrow_v.at[pl.ds(0, 1), pl.ds(q * VQ, min(VQ, V - q * VQ))],
                semr)
            for q in range(4)
        ]

    hrow = _fetch_row(j0)
    pltpu.sync_copy(ids_hbm.at[pl.ds(base, B_PER_W)], idx_v)

    bufs = (rows2a_v, rows2b_v)
    sems = (sem2a, sem2b)

    def _gather2(j):
        return pltpu.async_copy(
            w2_hbm.at[idx_v.at[pl.ds(j * CHUNK2, CHUNK2)]],
            bufs[j % 2], sems[j % 2])

    inflight = [_gather2(0), _gather2(1)]

    idxg = (idxga_v, idxgb_v)
    og = (oga_v, ogb_v)

    def _fetch_idx(c, p):
        return pltpu.async_copy(
            ids_hbm.at[pl.ds(c * IDX_CHUNK, IDX_CHUNK)], idxg[p], semi)

    hidx = [_fetch_idx(0, 0), None]
    row_ref = wrow_v.at[0]
    og_w = [None, None]
    w2w = [None, None]

    for h in hrow:
        h.wait()
    for r in range(R_PER_W):
        j = j0 + r
        if r > 0:
            for h in hrow:
                h.wait()
        for c in range(NIDX):
            p = c % 2
            hidx[p].wait()
            if c + 1 < NIDX:
                hidx[(c + 1) % 2] = _fetch_idx(c + 1, (c + 1) % 2)
            elif r + 1 < R_PER_W:
                hidx[0] = _fetch_idx(0, 0)
            if og_w[p] is not None:
                og_w[p].wait()
            if r == 0 and inflight is not None:
                inflight[p].wait()
                w2w[p] = pltpu.async_copy(
                    bufs[p], out2_hbm.at[pl.ds(base + c * CHUNK2, CHUNK2)],
                    semw)
            ochunk = og[p].at[0]
            ichunk = idxg[p]

            @plsc.parallel_loop(0, IDX_CHUNK // L, unroll=16)
            def _grp(g):
                iv = ichunk[pl.ds(g * L, L)]
                ochunk[pl.ds(g * L, L)] = plsc.load_gather(row_ref, [iv])

            og_w[p] = pltpu.async_copy(
                og[p],
                out1t_hbm.at[pl.ds(j, 1), pl.ds(c * IDX_CHUNK, IDX_CHUNK)],
                semo)
            if r == 0 and inflight is not None and c + 2 < NCHUNK2:
                w2w[p].wait()
                w2w[p] = None
                inflight[p] = _gather2(c + 2)
        if r + 1 < R_PER_W:
            hrow = _fetch_row(j0 + r + 1)
    for h in og_w + w2w:
        if h is not None:
            h.wait()


def kernel(instance_ids, W_instance, W_backgrounds):
    ids = jnp.squeeze(instance_ids).astype(jnp.int32)
    out1t, out2 = _dual_gather(ids, W_instance.T, W_backgrounds)
    return (out1t.T, out2)


# speedup vs baseline: 1.0185x; 1.0165x over previous
import functools

import jax
import jax.numpy as jnp
from jax import lax
from jax.experimental import pallas as pl
from jax.experimental.pallas import tpu as pltpu
from jax.experimental.pallas import tpu_sc as plsc

B = 16384
V = 100000
D1 = 64
D2 = 128
NC = 2
NS = 16
NW = NC * NS
B_PER_W = B // NW
CHUNK2 = 64
NCHUNK2 = B_PER_W // CHUNK2
R_PER_W = D1 // NW
IDX_CHUNK = 2048
NIDX = B // IDX_CHUNK
L = 16

_mesh = plsc.VectorSubcoreMesh(core_axis_name="c", subcore_axis_name="s")


@functools.partial(
    pl.kernel,
    mesh=_mesh,
    compiler_params=pltpu.CompilerParams(
        use_tc_tiling_on_sc=True, needs_layout_passes=False,
        internal_scratch_in_bytes=65536),
    out_type=(
        jax.ShapeDtypeStruct((D1, B), jnp.float32),
        jax.ShapeDtypeStruct((B, D2), jnp.float32),
    ),
    scratch_types=[
        pltpu.VMEM((B_PER_W,), jnp.int32),
        pltpu.VMEM((CHUNK2, D2), jnp.float32),
        pltpu.VMEM((CHUNK2, D2), jnp.float32),
        pltpu.VMEM((1, V), jnp.float32),
        pltpu.VMEM((IDX_CHUNK,), jnp.int32),
        pltpu.VMEM((IDX_CHUNK,), jnp.int32),
        pltpu.VMEM((1, IDX_CHUNK), jnp.float32),
        pltpu.VMEM((1, IDX_CHUNK), jnp.float32),
        pltpu.SemaphoreType.DMA,
        pltpu.SemaphoreType.DMA,
        pltpu.SemaphoreType.DMA,
        pltpu.SemaphoreType.DMA,
        pltpu.SemaphoreType.DMA,
        pltpu.SemaphoreType.DMA,
    ],
    name="sc_dual_gather",
)
def _dual_gather(ids_hbm, w1t_hbm, w2_hbm, out1t_hbm, out2_hbm,
                 idx_v, rows2a_v, rows2b_v, wrow_v,
                 idxga_v, idxgb_v, oga_v, ogb_v,
                 sem2a, sem2b, semr, semi, semo, semw):
    wid = lax.axis_index("s") * NC + lax.axis_index("c")
    base = wid * B_PER_W
    j0 = wid * R_PER_W

    hrow = pltpu.async_copy(w1t_hbm.at[pl.ds(j0, 1)], wrow_v, semr)
    pltpu.sync_copy(ids_hbm.at[pl.ds(base, B_PER_W)], idx_v)

    bufs = (rows2a_v, rows2b_v)
    sems = (sem2a, sem2b)

    def _gather2(j):
        return pltpu.async_copy(
            w2_hbm.at[idx_v.at[pl.ds(j * CHUNK2, CHUNK2)]],
            bufs[j % 2], sems[j % 2])

    inflight = [_gather2(0), _gather2(1)]

    idxg = (idxga_v, idxgb_v)
    og = (oga_v, ogb_v)

    def _fetch_idx(c, p):
        return pltpu.async_copy(
            ids_hbm.at[pl.ds(c * IDX_CHUNK, IDX_CHUNK)], idxg[p], semi)

    hidx = [_fetch_idx(0, 0), None]
    row_ref = wrow_v.at[0]
    og_w = [None, None]
    w2w = [None, None]

    hrow.wait()
    for r in range(R_PER_W):
        j = j0 + r
        if r > 0:
            hrow.wait()
        for c in range(NIDX):
            p = c % 2
            hidx[p].wait()
            if c + 1 < NIDX:
                hidx[(c + 1) % 2] = _fetch_idx(c + 1, (c + 1) % 2)
            elif r + 1 < R_PER_W:
                hidx[0] = _fetch_idx(0, 0)
            if og_w[p] is not None:
                og_w[p].wait()
            if r == 0:
                inflight[p].wait()
                w2w[p] = pltpu.async_copy(
                    bufs[p], out2_hbm.at[pl.ds(base + c * CHUNK2, CHUNK2)],
                    semw)
            ochunk = og[p].at[0]
            ichunk = idxg[p]

            @plsc.parallel_loop(0, IDX_CHUNK // L, unroll=8)
            def _grp(g):
                iv = ichunk[pl.ds(g * L, L)]
                ochunk[pl.ds(g * L, L)] = plsc.load_gather(row_ref, [iv])

            og_w[p] = pltpu.async_copy(
                og[p],
                out1t_hbm.at[pl.ds(j, 1), pl.ds(c * IDX_CHUNK, IDX_CHUNK)],
                semo)
            if r == 0 and c + 2 < NCHUNK2:
                w2w[p].wait()
                w2w[p] = None
                inflight[p] = _gather2(c + 2)
        if r + 1 < R_PER_W:
            hrow = pltpu.async_copy(
                w1t_hbm.at[pl.ds(j0 + r + 1, 1)], wrow_v, semr)
    for h in og_w + w2w:
        if h is not None:
            h.wait()


def kernel(instance_ids, W_instance, W_backgrounds):
    ids = jnp.squeeze(instance_ids).astype(jnp.int32)
    out1t, out2 = _dual_gather(ids, W_instance.T, W_backgrounds)
    return (out1t.T, out2)
